# baseline (speedup 1.0000x reference)
import functools

import jax
import jax.numpy as jnp
from jax.experimental import pallas as pl
from jax.experimental.pallas import tpu as pltpu
from jax.experimental.pallas import tpu_sc as plsc

_N = 8192
_E = 16
_K = 2
_BLW = 0.01
_BLK = 1024


def _dot_t(a, b):
    return jax.lax.dot_general(
        a, b, (((1,), (1,)), ((), ())),
        precision=jax.lax.Precision.DEFAULT,
        preferred_element_type=jnp.float32)


def _router_kernel(x_ref, w1_ref, b1_ref, w2_ref, b2_ref,
                   keys_ref, idx_ref, scr_ref, load_ref, bal_ref,
                   spe_acc):
    i = pl.program_id(0)
    nblocks = pl.num_programs(0)
    blk = x_ref.shape[0]
    e = keys_ref.shape[0]

    h1 = jnp.maximum(_dot_t(x_ref[...], w1_ref[...]) + b1_ref[...], 0.0)
    h = _dot_t(h1, w2_ref[...]) + b2_ref[...]
    logits = _dot_t(h, keys_ref[...])

    m = jnp.max(logits, axis=-1, keepdims=True)
    ex = jnp.exp(logits - m)
    denom = jnp.sum(ex, axis=-1, keepdims=True)
    p = ex / denom

    cols = jax.lax.broadcasted_iota(jnp.int32, (blk, e), 1)
    i1 = jnp.min(jnp.where(logits == m, cols, e), axis=-1, keepdims=True)
    masked = jnp.where(cols == i1, -jnp.inf, logits)
    m2 = jnp.max(masked, axis=-1, keepdims=True)
    i2 = jnp.min(jnp.where(masked == m2, cols, e), axis=-1, keepdims=True)

    oh1 = (cols == i1).astype(jnp.float32)
    oh2 = (cols == i2).astype(jnp.float32)
    p1 = jnp.sum(p * oh1, axis=-1, keepdims=True)
    p2 = jnp.sum(p * oh2, axis=-1, keepdims=True)
    tot = p1 + p2

    idx_ref[...] = jnp.concatenate([i1, i2], axis=1)
    scr_ref[...] = jnp.concatenate([p1 / tot, p2 / tot], axis=1)

    tpe_blk = jnp.sum(oh1 + oh2, axis=0, keepdims=True)
    spe_blk = jnp.sum(p, axis=0, keepdims=True)

    @pl.when(i == 0)
    def _init():
        load_ref[...] = tpe_blk
        spe_acc[...] = spe_blk

    @pl.when(i != 0)
    def _acc():
        load_ref[...] += tpe_blk
        spe_acc[...] += spe_blk

    @pl.when(i == nblocks - 1)
    def _fin():
        n = jnp.float32(_N)
        tpe = load_ref[...] / n
        spe = spe_acc[...] / n
        bal_ref[...] = jnp.sum(tpe * spe).reshape(1, 1) * (_BLW * _E)


def _router_tc(x, W1, b1r, W2, b2r, keys):
    n, d_in = x.shape
    d_hid = W1.shape[0]
    d_out = W2.shape[0]
    e = keys.shape[0]
    blk = _BLK
    grid = n // blk

    out_shapes = (
        jax.ShapeDtypeStruct((n, _K), jnp.int32),
        jax.ShapeDtypeStruct((n, _K), jnp.float32),
        jax.ShapeDtypeStruct((1, e), jnp.float32),
        jax.ShapeDtypeStruct((1, 1), jnp.float32),
    )
    in_specs = [
        pl.BlockSpec((blk, d_in), lambda i: (i, 0)),
        pl.BlockSpec((d_hid, d_in), lambda i: (0, 0)),
        pl.BlockSpec((1, d_hid), lambda i: (0, 0)),
        pl.BlockSpec((d_out, d_hid), lambda i: (0, 0)),
        pl.BlockSpec((1, d_out), lambda i: (0, 0)),
        pl.BlockSpec((e, d_out), lambda i: (0, 0)),
    ]
    out_specs = (
        pl.BlockSpec((blk, _K), lambda i: (i, 0)),
        pl.BlockSpec((blk, _K), lambda i: (i, 0)),
        pl.BlockSpec((1, e), lambda i: (0, 0)),
        pl.BlockSpec((1, 1), lambda i: (0, 0)),
    )
    return pl.pallas_call(
        _router_kernel,
        grid=(grid,),
        in_specs=in_specs,
        out_specs=out_specs,
        out_shape=out_shapes,
        scratch_shapes=[pltpu.VMEM((1, e), jnp.float32)],
    )(x, W1, b1r, W2, b2r, keys)


_SC_CHUNK = 32


def _sc_gather(keys, idx_flat):
    n_rows = idx_flat.shape[0]
    d = keys.shape[1]
    info = plsc.get_sparse_core_info()
    nc, ns = info.num_cores, info.num_subcores
    nw = nc * ns
    b_per_w = n_rows // nw
    chunk = _SC_CHUNK
    nchunks = b_per_w // chunk
    mesh = plsc.VectorSubcoreMesh(core_axis_name="c", subcore_axis_name="s")

    @functools.partial(
        pl.kernel, mesh=mesh,
        out_type=jax.ShapeDtypeStruct((n_rows, d), jnp.float32),
        scratch_types=[
            pltpu.VMEM((b_per_w,), jnp.int32),
            pltpu.VMEM((chunk, d), jnp.float32),
            pltpu.VMEM((chunk, d), jnp.float32),
            pltpu.SemaphoreType.DMA,
            pltpu.SemaphoreType.DMA,
            pltpu.SemaphoreType.DMA,
            pltpu.SemaphoreType.DMA,
        ],
    )
    def gather_kernel(keys_hbm, idx_hbm, out_hbm, idx_v, buf0, buf1,
                      gsem0, gsem1, ssem0, ssem1):
        wid = jax.lax.axis_index("s") * nc + jax.lax.axis_index("c")
        base = wid * b_per_w
        pltpu.sync_copy(idx_hbm.at[pl.ds(base, b_per_w)], idx_v)
        bufs = (buf0, buf1)
        gsems = (gsem0, gsem1)
        ssems = (ssem0, ssem1)

        gathers = [None, None]
        scatters = [None, None]
        for c in range(nchunks):
            s = c % 2
            if scatters[s] is not None:
                scatters[s].wait()
                scatters[s] = None
            gathers[s] = pltpu.async_copy(
                keys_hbm.at[idx_v.at[pl.ds(c * chunk, chunk)]],
                bufs[s], gsems[s])
            if c > 0:
                sp = (c - 1) % 2
                gathers[sp].wait()
                scatters[sp] = pltpu.async_copy(
                    bufs[sp],
                    out_hbm.at[pl.ds(base + (c - 1) * chunk, chunk)],
                    ssems[sp])
        last = nchunks - 1
        s = last % 2
        gathers[s].wait()
        scatters[s] = pltpu.async_copy(
            bufs[s], out_hbm.at[pl.ds(base + last * chunk, chunk)], ssems[s])
        for sc in scatters:
            if sc is not None:
                sc.wait()

    return gather_kernel(keys, idx_flat)


@functools.partial(jax.jit, static_argnames=("interpret",))
def kernel(x, W1, b1, W2, b2, keys, interpret=False):
    n = x.shape[0]
    d_out = W2.shape[0]
    e = keys.shape[0]

    b1r = b1[None, :]
    b2r = b2[None, :]

    idx, scr, load2d, bal = _router_tc(x, W1, b1r, W2, b2r, keys)

    idx_flat = idx.reshape(n * _K)
    sel_flat = _sc_gather(keys, idx_flat)
    selected_keys = sel_flat.reshape(n, _K, d_out)

    top_k_indices = idx
    top_k_scores = scr
    load = load2d.reshape(e)
    balance_loss = bal.reshape(())
    importance = jnp.float32(0.0)
    return (top_k_indices, top_k_scores, balance_loss, load, importance,
            selected_keys)


# speedup vs baseline: 6.6653x; 6.6653x over previous
import functools

import jax
import jax.numpy as jnp
from jax.experimental import pallas as pl
from jax.experimental.pallas import tpu as pltpu

_N = 8192
_E = 16
_K = 2
_BLW = 0.01
_BLK = 1024


def _dot_t(a, b):
    return jax.lax.dot_general(
        a, b, (((1,), (1,)), ((), ())),
        precision=jax.lax.Precision.DEFAULT,
        preferred_element_type=jnp.float32)


def _router_kernel(x_ref, w1_ref, b1_ref, w2_ref, b2_ref,
                   keys_ref, idx_ref, scr_ref, sel_ref, load_ref, bal_ref,
                   spe_acc):
    i = pl.program_id(0)
    nblocks = pl.num_programs(0)
    blk = x_ref.shape[0]
    e = keys_ref.shape[0]
    d_out = keys_ref.shape[1]

    hp = jax.lax.Precision.DEFAULT
    h1 = jnp.maximum(_dot_t(x_ref[...], w1_ref[...]) + b1_ref[...], 0.0)
    h = _dot_t(h1, w2_ref[...]) + b2_ref[...]
    logits = _dot_t(h, keys_ref[...])

    m = jnp.max(logits, axis=-1, keepdims=True)
    ex = jnp.exp(logits - m)
    denom = jnp.sum(ex, axis=-1, keepdims=True)
    p = ex / denom

    cols = jax.lax.broadcasted_iota(jnp.int32, (blk, e), 1)
    i1 = jnp.min(jnp.where(logits == m, cols, e), axis=-1, keepdims=True)
    masked = jnp.where(cols == i1, -jnp.inf, logits)
    m2 = jnp.max(masked, axis=-1, keepdims=True)
    i2 = jnp.min(jnp.where(masked == m2, cols, e), axis=-1, keepdims=True)

    oh1 = (cols == i1).astype(jnp.float32)
    oh2 = (cols == i2).astype(jnp.float32)
    p1 = jnp.sum(p * oh1, axis=-1, keepdims=True)
    p2 = jnp.sum(p * oh2, axis=-1, keepdims=True)
    tot = p1 + p2

    idx_ref[...] = jnp.concatenate([i1, i2], axis=1)
    scr_ref[...] = jnp.concatenate([p1 / tot, p2 / tot], axis=1)

    kb = keys_ref[...].astype(jnp.bfloat16)
    sel_ref[:, 0, :] = jnp.dot(oh1.astype(jnp.bfloat16), kb,
                               preferred_element_type=jnp.float32)
    sel_ref[:, 1, :] = jnp.dot(oh2.astype(jnp.bfloat16), kb,
                               preferred_element_type=jnp.float32)

    tpe_blk = jnp.sum(oh1 + oh2, axis=0, keepdims=True)
    spe_blk = jnp.sum(p, axis=0, keepdims=True)

    @pl.when(i == 0)
    def _init():
        load_ref[...] = tpe_blk
        spe_acc[...] = spe_blk

    @pl.when(i != 0)
    def _acc():
        load_ref[...] += tpe_blk
        spe_acc[...] += spe_blk

    @pl.when(i == nblocks - 1)
    def _fin():
        n = jnp.float32(_N)
        tpe = load_ref[...] / n
        spe = spe_acc[...] / n
        bal_ref[...] = jnp.sum(tpe * spe).reshape(1, 1) * (_BLW * _E)


@functools.partial(jax.jit, static_argnames=("interpret",))
def kernel(x, W1, b1, W2, b2, keys, interpret=False):
    n, d_in = x.shape
    d_hid = W1.shape[0]
    d_out = W2.shape[0]
    e = keys.shape[0]
    blk = _BLK
    grid = n // blk

    b1r = b1[None, :]
    b2r = b2[None, :]

    out_shapes = (
        jax.ShapeDtypeStruct((n, _K), jnp.int32),
        jax.ShapeDtypeStruct((n, _K), jnp.float32),
        jax.ShapeDtypeStruct((n, _K, d_out), jnp.float32),
        jax.ShapeDtypeStruct((1, e), jnp.float32),
        jax.ShapeDtypeStruct((1, 1), jnp.float32),
    )
    in_specs = [
        pl.BlockSpec((blk, d_in), lambda i: (i, 0)),
        pl.BlockSpec((d_hid, d_in), lambda i: (0, 0)),
        pl.BlockSpec((1, d_hid), lambda i: (0, 0)),
        pl.BlockSpec((d_out, d_hid), lambda i: (0, 0)),
        pl.BlockSpec((1, d_out), lambda i: (0, 0)),
        pl.BlockSpec((e, d_out), lambda i: (0, 0)),
    ]
    out_specs = (
        pl.BlockSpec((blk, _K), lambda i: (i, 0)),
        pl.BlockSpec((blk, _K), lambda i: (i, 0)),
        pl.BlockSpec((blk, _K, d_out), lambda i: (i, 0, 0)),
        pl.BlockSpec((1, e), lambda i: (0, 0)),
        pl.BlockSpec((1, 1), lambda i: (0, 0)),
    )

    idx, scr, sel, load2d, bal = pl.pallas_call(
        _router_kernel,
        grid=(grid,),
        in_specs=in_specs,
        out_specs=out_specs,
        out_shape=out_shapes,
        scratch_shapes=[pltpu.VMEM((1, e), jnp.float32)],
        interpret=interpret,
    )(x, W1, b1r, W2, b2r, keys)

    top_k_indices = idx
    top_k_scores = scr
    selected_keys = sel
    load = load2d.reshape(e)
    balance_loss = bal.reshape(())
    importance = jnp.float32(0.0)
    return (top_k_indices, top_k_scores, balance_loss, load, importance,
            selected_keys)
